# baseline (speedup 1.0000x reference)
import jax
import jax.numpy as jnp
from jax import lax
from jax.experimental import pallas as pl
from jax.experimental.pallas import tpu as pltpu
from jax.experimental.pallas import tpu_sc as plsc

_BATCH = 128
_N = 32768
_L = 16
_NSTEP = _N // _L
_NW = 32
_ROWS_PER_W = _BATCH // _NW
_HBINS = 4096


def _f32_to_key(x):
    i = plsc.bitcast(x, jnp.int32)
    u = plsc.bitcast(x, jnp.uint32)
    s = lax.shift_right_arithmetic(i, 31)
    flip = plsc.bitcast(s, jnp.uint32) | jnp.uint32(0x80000000)
    return u ^ flip


def _key_to_f32(k):
    neg = k < jnp.uint32(0x80000000)
    u = jnp.where(neg, ~k, k ^ jnp.uint32(0x80000000))
    return plsc.bitcast(u, jnp.float32)


def _search(hist_v, nbins, kv):
    def step(i, carry):
        carry_s, bacc, wacc = carry
        h = hist_v[pl.ds(i * _L, _L)]
        run = carry_s + plsc.cumsum(h)
        le = run <= kv
        bacc = bacc + jnp.where(le, 1, 0)
        wacc = wacc + jnp.where(le, h, 0)
        return carry_s + jnp.sum(h), bacc, wacc

    zero = jnp.zeros((_L,), jnp.int32)
    _, bacc, wacc = lax.fori_loop(
        0, nbins // _L, step, (jnp.int32(0), zero, zero), unroll=4)
    return jnp.sum(bacc), jnp.sum(wacc)


def _zero(hist_v, nbins):
    def z(i, _):
        hist_v[pl.ds(i * _L, _L)] = jnp.zeros((_L,), jnp.int32)
        return 0
    lax.fori_loop(0, nbins // _L, z, 0, unroll=8)


def _sc_body(x_hbm, probs_hbm, out_hbm, row_v, key_v, hist_v, probs_v):
    c = lax.axis_index("c")
    s = lax.axis_index("s")
    wid = s * 2 + c
    pltpu.sync_copy(probs_hbm, probs_v)
    ones = jnp.ones((_L,), jnp.int32)

    def row_body(j, _):
        row = wid * _ROWS_PER_W + j
        pltpu.sync_copy(x_hbm.at[row], row_v)

        p16 = probs_v[pl.ds((row // _L) * _L, _L)]
        sel = lax.iota(jnp.int32, _L) == (row % _L)
        p = jnp.sum(jnp.where(sel, p16, 0.0))
        v = jnp.float32(_N) * p
        vi = v.astype(jnp.int32)
        ceil_v = vi + jnp.where(vi.astype(jnp.float32) < v, 1, 0)
        kv = jnp.clip(ceil_v - 1, 0, _N - 1)

        _zero(hist_v, _HBINS)

        def l0(i, _):
            x = row_v[pl.ds(i * _L, _L)]
            key = _f32_to_key(x)
            key_v[pl.ds(i * _L, _L)] = key
            b = (key >> jnp.uint32(20)).astype(jnp.int32)
            plsc.addupdate_scatter(hist_v, [b], ones)
            return 0

        lax.fori_loop(0, _NSTEP, l0, 0, unroll=8)
        b0, below0 = _search(hist_v, _HBINS, kv)
        kv = kv - below0
        pfx0 = b0.astype(jnp.uint32)

        _zero(hist_v, _HBINS)

        def l1(i, _):
            key = key_v[pl.ds(i * _L, _L)]
            m = (key >> jnp.uint32(20)) == pfx0
            b = ((key >> jnp.uint32(8)) & jnp.uint32(0xFFF)).astype(jnp.int32)
            plsc.addupdate_scatter(hist_v, [b], ones, mask=m)
            return 0

        lax.fori_loop(0, _NSTEP, l1, 0, unroll=8)
        b1, below1 = _search(hist_v, _HBINS, kv)
        kv = kv - below1
        pfx1 = (pfx0 << jnp.uint32(12)) | b1.astype(jnp.uint32)

        _zero(hist_v, 256)

        def l2(i, _):
            key = key_v[pl.ds(i * _L, _L)]
            m = (key >> jnp.uint32(8)) == pfx1
            b = (key & jnp.uint32(0xFF)).astype(jnp.int32)
            plsc.addupdate_scatter(hist_v, [b], ones, mask=m)
            return 0

        lax.fori_loop(0, _NSTEP, l2, 0, unroll=8)
        b2, _unused = _search(hist_v, 256, kv)
        tkey = (pfx1 << jnp.uint32(8)) | b2.astype(jnp.uint32)

        thresh = _key_to_f32(tkey + jnp.zeros((_L,), jnp.uint32))

        def mask_step(i, _, thresh=thresh):
            x = row_v[pl.ds(i * _L, _L)]
            row_v[pl.ds(i * _L, _L)] = jnp.where(x >= thresh, x, 0.0)
            return 0

        lax.fori_loop(0, _NSTEP, mask_step, 0, unroll=8)
        pltpu.sync_copy(row_v, out_hbm.at[row])
        return 0

    lax.fori_loop(0, _ROWS_PER_W, row_body, 0)


_sc_masking = pl.kernel(
    _sc_body,
    out_type=jax.ShapeDtypeStruct((_BATCH, _N), jnp.float32),
    mesh=plsc.VectorSubcoreMesh(core_axis_name="c", subcore_axis_name="s"),
    compiler_params=pltpu.CompilerParams(needs_layout_passes=False),
    scratch_types=[
        pltpu.VMEM((_N,), jnp.float32),
        pltpu.VMEM((_N,), jnp.uint32),
        pltpu.VMEM((_HBINS,), jnp.int32),
        pltpu.VMEM((_BATCH,), jnp.float32),
    ],
)


@jax.jit
def _masked(inputs, probs):
    return _sc_masking(inputs, probs)


def kernel(inputs, probs, training=True):
    out = _masked(inputs, probs)
    try:
        static_training = bool(training)
    except jax.errors.TracerBoolConversionError:
        return jnp.where(training, out, inputs)
    return out if static_training else inputs
